# baseline (speedup 1.0000x reference)
import functools

import jax
import jax.numpy as jnp
from jax import lax
from jax.experimental import pallas as pl
from jax.experimental.pallas import tpu as pltpu
from jax.experimental.pallas import tpu_sc as plsc

D_PROJ = 128
SCALE = float(D_PROJ) ** 0.5

R0, N0 = 10000, 20000
R1, N1 = 16000, 80000
R2, N2 = 16384, 901120
S1 = 32000
S2 = 114688
NTAB = S2 + N2

T = 4096 * 200
CHUNK = 128
NBUF = 5
LEAD = 4


def _region_call(body, grid, in_specs, out_spec, table=None, extra=()):
    kwargs = {}
    ins = ()
    if table is not None:
        ins = (table,)
        in_specs = [pl.BlockSpec(memory_space=pl.ANY)] + in_specs
        kwargs["input_output_aliases"] = {0: 0}
    return pl.pallas_call(
        body,
        grid=(grid,),
        in_specs=in_specs,
        out_specs=out_spec,
        out_shape=jax.ShapeDtypeStruct((NTAB, D_PROJ), jnp.float32),
        **kwargs,
    )(*ins, *extra)


def _mm(e_ref, p_ref, out_ref, cdim):
    out_ref[...] = lax.dot_general(
        e_ref[...], p_ref[...],
        dimension_numbers=(((cdim,), (0,)), ((), ())),
        preferred_element_type=jnp.float32,
    )


def _body0(e_ref, p_ref, out_ref):
    _mm(e_ref, p_ref, out_ref, 1)


def _body12(tab_ref, e_ref, p_ref, out_ref):
    del tab_ref
    _mm(e_ref, p_ref, out_ref, 0)


def _build_table(emb0, emb1, emb2, proj0, proj1, proj2):
    p0t = SCALE * proj0.T
    p1t = SCALE * proj1.T
    p2t = SCALE * proj2.T
    emb1t = emb1.T
    emb2t = emb2.T

    tab = _region_call(
        _body0, N0 // R0,
        [pl.BlockSpec((R0, 128), lambda g: (g, 0)),
         pl.BlockSpec((128, 128), lambda g: (0, 0))],
        pl.BlockSpec((R0, D_PROJ), lambda g: (g, 0)),
        extra=(emb0, p0t))
    tab = _region_call(
        _body12, N1 // R1,
        [pl.BlockSpec((32, R1), lambda g: (0, g)),
         pl.BlockSpec((32, 128), lambda g: (0, 0))],
        pl.BlockSpec((R1, D_PROJ), lambda g: (S1 // R1 + g, 0)),
        table=tab, extra=(emb1t, p1t))
    tab = _region_call(
        _body12, N2 // R2,
        [pl.BlockSpec((8, R2), lambda g: (0, g)),
         pl.BlockSpec((8, 128), lambda g: (0, 0))],
        pl.BlockSpec((R2, D_PROJ), lambda g: (S2 // R2 + g, 0)),
        table=tab, extra=(emb2t, p2t))
    return tab


def _make_gather():
    info = plsc.get_sparse_core_info()
    nw = info.num_cores * info.num_subcores
    tpw = T // nw
    nchunk = tpw // CHUNK

    mesh = plsc.VectorSubcoreMesh(core_axis_name="c", subcore_axis_name="s")

    @functools.partial(
        pl.kernel,
        mesh=mesh,
        out_type=jax.ShapeDtypeStruct((T, D_PROJ), jnp.float32),
        scratch_types=(
            [pltpu.VMEM((nchunk, CHUNK), jnp.int32)]
            + [pltpu.VMEM((CHUNK, D_PROJ), jnp.float32) for _ in range(NBUF)]
            + [pltpu.SemaphoreType.DMA for _ in range(2 * NBUF)]
        ),
    )
    def gather_kernel(p_hbm, idx_hbm, out_hbm, idx_v, *bufs_and_sems):
        rows = bufs_and_sems[:NBUF]
        sems_g = bufs_and_sems[NBUF:2 * NBUF]
        sems_s = bufs_and_sems[2 * NBUF:]
        wid = lax.axis_index("s") * info.num_cores + lax.axis_index("c")
        base = wid * tpw
        pltpu.sync_copy(idx_hbm.at[wid], idx_v)

        def drain_scatter(sem):
            pltpu.make_async_copy(
                rows[0], out_hbm.at[pl.ds(base, CHUNK)], sem).wait()

        def body(g, carry):
            @pl.when(g < nchunk)
            def _():
                for b in range(NBUF):
                    @pl.when(g % NBUF == b)
                    def _(b=b):
                        @pl.when(g >= NBUF)
                        def _():
                            drain_scatter(sems_s[b])
                        pltpu.async_copy(p_hbm.at[idx_v.at[g]], rows[b], sems_g[b])

            @pl.when(g >= LEAD)
            def _():
                gp = g - LEAD
                for b in range(NBUF):
                    @pl.when(gp % NBUF == b)
                    def _(b=b, gp=gp):
                        pltpu.make_async_copy(
                            p_hbm.at[idx_v.at[gp]], rows[b], sems_g[b]).wait()
                        pltpu.async_copy(
                            rows[b], out_hbm.at[pl.ds(base + gp * CHUNK, CHUNK)],
                            sems_s[b])
            return carry

        lax.fori_loop(0, nchunk + LEAD, body, 0)
        for b in range(NBUF):
            drain_scatter(sems_s[b])

    return gather_kernel, nw, nchunk


def kernel(inp, emb0, emb1, emb2, proj0, proj1, proj2):
    gather_kernel, nw, nchunk = _make_gather()
    table = _build_table(emb0, emb1, emb2, proj0, proj1, proj2)
    tok = inp.reshape(-1).astype(jnp.int32)
    idx = tok + jnp.where(tok >= 100000, S2 - 100000,
                          jnp.where(tok >= 20000, S1 - 20000, 0)).astype(jnp.int32)
    out = gather_kernel(table, idx.reshape(nw, nchunk, CHUNK))
    return out.reshape(inp.shape + (D_PROJ,))
